# Initial kernel scaffold; baseline (speedup 1.0000x reference)
import jax
import jax.numpy as jnp
from jax.experimental import pallas as pl


def kernel(words, feats, adds, pos, s_arc, a_arc):
    raise NotImplementedError("write your pallas kernel here")


# speedup vs baseline: 1010.2688x; 1010.2688x over previous
import functools

import jax
import jax.numpy as jnp
from jax.experimental import pallas as pl

N_POS = 50
ALPHA = 0.3
NP = 128

_HIGHEST = jax.lax.Precision.HIGHEST


def _hist_body(adds_ref, a_ref, out_ref):
    b = pl.program_id(0)
    adds_row = adds_ref[0]
    qs = jax.lax.broadcasted_iota(jnp.int32, (NP, adds_row.shape[1]), 0)
    et = (qs == adds_row).astype(jnp.float32)
    a_b = a_ref[0]
    t1 = jax.lax.dot_general(et, a_b, (((1,), (0,)), ((), ())),
                             preferred_element_type=jnp.float32,
                             precision=_HIGHEST)
    g = jax.lax.dot_general(t1, et, (((1,), (1,)), ((), ())),
                            preferred_element_type=jnp.float32,
                            precision=_HIGHEST)

    @pl.when(b == 0)
    def _():
        out_ref[...] = jnp.zeros_like(out_ref)

    out_ref[...] += g


def _apply_body(pos_ref, g_ref, s_ref, out_ref):
    pos_row = pos_ref[0]
    qs = jax.lax.broadcasted_iota(jnp.int32, (NP, pos_row.shape[1]), 0)
    pt = (qs == pos_row).astype(jnp.float32)
    gs = jax.nn.sigmoid(g_ref[...])
    t = jax.lax.dot_general(gs, pt, (((1,), (0,)), ((), ())),
                            preferred_element_type=jnp.float32,
                            precision=_HIGHEST)
    add = jax.lax.dot_general(pt, t, (((0,), (0,)), ((), ())),
                              preferred_element_type=jnp.float32,
                              precision=_HIGHEST)
    out_ref[0] = s_ref[0] + ALPHA * add


@jax.jit
def kernel(words, feats, adds, pos, s_arc, a_arc):
    del words, feats
    B, S = adds.shape
    adds3 = adds.reshape(B, 1, S)
    pos3 = pos.reshape(B, 1, S)

    g = pl.pallas_call(
        _hist_body,
        grid=(B,),
        in_specs=[
            pl.BlockSpec((1, 1, S), lambda b: (b, 0, 0)),
            pl.BlockSpec((1, S, S), lambda b: (b, 0, 0)),
        ],
        out_specs=pl.BlockSpec((NP, NP), lambda b: (0, 0)),
        out_shape=jax.ShapeDtypeStruct((NP, NP), jnp.float32),
    )(adds3, a_arc)

    out = pl.pallas_call(
        _apply_body,
        grid=(B,),
        in_specs=[
            pl.BlockSpec((1, 1, S), lambda b: (b, 0, 0)),
            pl.BlockSpec((NP, NP), lambda b: (0, 0)),
            pl.BlockSpec((1, S, S), lambda b: (b, 0, 0)),
        ],
        out_specs=pl.BlockSpec((1, S, S), lambda b: (b, 0, 0)),
        out_shape=jax.ShapeDtypeStruct((B, S, S), jnp.float32),
    )(pos3, g, s_arc)
    return out


# speedup vs baseline: 1290.5014x; 1.2774x over previous
import functools

import jax
import jax.numpy as jnp
from jax.experimental import pallas as pl

N_POS = 50
ALPHA = 0.3
NP = 64


def _split(x):
    hi = x.astype(jnp.bfloat16)
    lo = (x - hi.astype(jnp.float32)).astype(jnp.bfloat16)
    return hi, lo


def _dot_exact(onehot_bf16, dense_f32, dnums):
    hi, lo = _split(dense_f32)
    d = functools.partial(jax.lax.dot_general, dimension_numbers=dnums,
                          preferred_element_type=jnp.float32)
    return d(onehot_bf16, hi) + d(onehot_bf16, lo)


def _hist_body(adds_ref, a_ref, out_ref):
    b = pl.program_id(0)
    adds_row = adds_ref[0]
    qs = jax.lax.broadcasted_iota(jnp.int32, (NP, adds_row.shape[1]), 0)
    et = (qs == adds_row).astype(jnp.bfloat16)
    t1 = _dot_exact(et, a_ref[0], (((1,), (0,)), ((), ())))
    gt = _dot_exact(et, t1, (((1,), (1,)), ((), ())))

    @pl.when(b == 0)
    def _():
        out_ref[...] = jnp.zeros_like(out_ref)

    out_ref[...] += gt


def _apply_body(pos_ref, g_ref, s_ref, out_ref):
    pos_row = pos_ref[0]
    qs = jax.lax.broadcasted_iota(jnp.int32, (NP, pos_row.shape[1]), 0)
    pt = (qs == pos_row).astype(jnp.bfloat16)
    gs = jax.nn.sigmoid(g_ref[...])
    w = _dot_exact(pt, gs, (((0,), (0,)), ((), ())))
    add = _dot_exact(pt, w, (((0,), (1,)), ((), ())))
    out_ref[0] = s_ref[0] + ALPHA * add


@jax.jit
def kernel(words, feats, adds, pos, s_arc, a_arc):
    del words, feats
    B, S = adds.shape
    adds3 = adds.reshape(B, 1, S)
    pos3 = pos.reshape(B, 1, S)

    g = pl.pallas_call(
        _hist_body,
        grid=(B,),
        in_specs=[
            pl.BlockSpec((1, 1, S), lambda b: (b, 0, 0)),
            pl.BlockSpec((1, S, S), lambda b: (b, 0, 0)),
        ],
        out_specs=pl.BlockSpec((NP, NP), lambda b: (0, 0)),
        out_shape=jax.ShapeDtypeStruct((NP, NP), jnp.float32),
    )(adds3, a_arc)

    out = pl.pallas_call(
        _apply_body,
        grid=(B,),
        in_specs=[
            pl.BlockSpec((1, 1, S), lambda b: (b, 0, 0)),
            pl.BlockSpec((NP, NP), lambda b: (0, 0)),
            pl.BlockSpec((1, S, S), lambda b: (b, 0, 0)),
        ],
        out_specs=pl.BlockSpec((1, S, S), lambda b: (b, 0, 0)),
        out_shape=jax.ShapeDtypeStruct((B, S, S), jnp.float32),
    )(pos3, g, s_arc)
    return out


# speedup vs baseline: 2109.4366x; 1.6346x over previous
import functools

import jax
import jax.numpy as jnp
from jax.experimental import pallas as pl

N_POS = 50
ALPHA = 0.3
NP = 64
KH = 8
KA = 4


def _split(x):
    hi = x.astype(jnp.bfloat16)
    lo = (x - hi.astype(jnp.float32)).astype(jnp.bfloat16)
    return hi, lo


def _dot_exact(onehot_bf16, dense_f32, dnums):
    hi, lo = _split(dense_f32)
    d = functools.partial(jax.lax.dot_general, dimension_numbers=dnums,
                          preferred_element_type=jnp.float32)
    return d(onehot_bf16, hi) + d(onehot_bf16, lo)


def _hist_body(*refs):
    out_ref = refs[-1]
    adds_refs, a_refs = refs[:KH], refs[KH:2 * KH]
    b = pl.program_id(0)
    acc = jnp.zeros((NP, NP), jnp.float32)
    for adds_r, a_ref in zip(adds_refs, a_refs):
        adds_row = adds_r[0]
        qs = jax.lax.broadcasted_iota(jnp.int32, (NP, adds_row.shape[1]), 0)
        et = (qs == adds_row).astype(jnp.bfloat16)
        t1 = _dot_exact(et, a_ref[0], (((1,), (0,)), ((), ())))
        acc = acc + _dot_exact(et, t1, (((1,), (1,)), ((), ())))

    @pl.when(b == 0)
    def _():
        out_ref[...] = jnp.zeros_like(out_ref)

    out_ref[...] += acc


def _apply_body(*refs):
    pos_ref, g_ref = refs[0], refs[1]
    s_refs, out_ref = refs[2:2 + KA], refs[-1]
    gs = jax.nn.sigmoid(g_ref[...])
    for k, s_ref in enumerate(s_refs):
        pos_row = pos_ref[k]
        qs = jax.lax.broadcasted_iota(jnp.int32, (NP, pos_row.shape[1]), 0)
        pt = (qs == pos_row).astype(jnp.bfloat16)
        w = _dot_exact(pt, gs, (((0,), (0,)), ((), ())))
        add = _dot_exact(pt, w, (((0,), (1,)), ((), ())))
        out_ref[k] = s_ref[0] + ALPHA * add


@jax.jit
def kernel(words, feats, adds, pos, s_arc, a_arc):
    del words, feats
    B, S = adds.shape
    adds3 = adds.reshape(B, 1, S)
    pos3 = pos.reshape(B, 1, S)

    H = B // KH
    hmaps = [(lambda b, k=k: (b + k * H, 0, 0)) for k in range(KH)]
    g = pl.pallas_call(
        _hist_body,
        grid=(H,),
        in_specs=[pl.BlockSpec((1, 1, S), m) for m in hmaps]
                 + [pl.BlockSpec((1, S, S), m) for m in hmaps],
        out_specs=pl.BlockSpec((NP, NP), lambda b: (0, 0)),
        out_shape=jax.ShapeDtypeStruct((NP, NP), jnp.float32),
    )(*([adds3] * KH), *([a_arc] * KH))

    amaps = [(lambda b, k=k: (KA * b + k, 0, 0)) for k in range(KA)]
    out = pl.pallas_call(
        _apply_body,
        grid=(B // KA,),
        in_specs=[
            pl.BlockSpec((KA, 1, S), lambda b: (b, 0, 0)),
            pl.BlockSpec((NP, NP), lambda b: (0, 0)),
        ] + [pl.BlockSpec((1, S, S), m) for m in amaps],
        out_specs=pl.BlockSpec((KA, S, S), lambda b: (b, 0, 0)),
        out_shape=jax.ShapeDtypeStruct((B, S, S), jnp.float32),
    )(pos3, g, *([s_arc] * KA))
    return out
